# baseline (speedup 1.0000x reference)
import functools

import jax
import jax.numpy as jnp
import numpy as np
from jax import lax
from jax.experimental import pallas as pl
from jax.experimental.pallas import tpu as pltpu
from jax.experimental.pallas import tpu_sc as plsc

V = 100000
D = 128
B = 500000

NC, NS = 2, 16
NW = NC * NS
C = 200
SUB = 100
KSUB = C // SUB
NCHUNK = B // C
NBUF = 4
CPW = (NCHUNK + NW - 1) // NW

_ADD_BLOCK = 2000


def _add_body(a_ref, b_ref, o_ref):
    o_ref[...] = a_ref[...] + b_ref[...]


def _fuse_tables(nf, mem):
    return pl.pallas_call(
        _add_body,
        grid=(V // _ADD_BLOCK,),
        in_specs=[pl.BlockSpec((_ADD_BLOCK, D), lambda i: (i, 0))] * 2,
        out_specs=pl.BlockSpec((_ADD_BLOCK, D), lambda i: (i, 0)),
        out_shape=jax.ShapeDtypeStruct((V, D), jnp.float32),
    )(nf, mem)


_PERM = np.zeros((NW, CPW), dtype=np.int32)
for _w in range(NW):
    _js = np.arange(_w, NCHUNK, NW, dtype=np.int32)
    _PERM[_w, : len(_js)] = _js
_PERM_J = _PERM.reshape(-1)

_MESH = plsc.VectorSubcoreMesh(
    core_axis_name="c", subcore_axis_name="s", num_cores=NC, num_subcores=NS
)


@functools.partial(
    pl.kernel,
    out_type=jax.ShapeDtypeStruct((B, D), jnp.float32),
    mesh=_MESH,
    scratch_types=[
        pltpu.VMEM((CPW, KSUB, SUB), jnp.int32),
        *[pltpu.VMEM((C, D), jnp.float32) for _ in range(NBUF)],
        *[pltpu.SemaphoreType.DMA for _ in range(2 * NBUF)],
    ],
)
def _gather_k(table_hbm, idx_hbm, out_hbm, idx_all, *bufs_and_sems):
    rows = bufs_and_sems[:NBUF]
    sg = bufs_and_sems[NBUF : 2 * NBUF]
    ss = bufs_and_sems[2 * NBUF :]
    wid = lax.axis_index("s") * NC + lax.axis_index("c")
    n_mine = (NCHUNK - wid + NW - 1) // NW

    pltpu.sync_copy(idx_hbm.at[wid], idx_all)

    def gather_copies(i, rowsb, semg, make_only):
        mk = pltpu.make_async_copy if make_only else pltpu.async_copy
        return [
            mk(
                table_hbm.at[idx_all.at[i, k]],
                rowsb.at[pl.ds(k * SUB, SUB), :],
                semg,
            )
            for k in range(KSUB)
        ]

    for b in range(NBUF):
        gather_copies(b, rows[b], sg[b], make_only=False)

    def body(g, carry):
        for b in range(NBUF):
            t = NBUF * g + b

            @pl.when(t < n_mine)
            def _process():
                for cp in gather_copies(t, rows[b], sg[b], make_only=True):
                    cp.wait()
                j = wid + t * NW
                st = pltpu.async_copy(rows[b], out_hbm.at[pl.ds(j * C, C), :], ss[b])
                st.wait()

                @pl.when(t + NBUF < n_mine)
                def _refill():
                    gather_copies(t + NBUF, rows[b], sg[b], make_only=False)

        return carry

    lax.fori_loop(0, (n_mine + NBUF - 1) // NBUF, body, 0)


def kernel(node_features, memory, source_nodes, timestamps, time_w, time_b):
    del timestamps, time_w, time_b
    fused = _fuse_tables(node_features, memory)
    idx = source_nodes.astype(jnp.int32).reshape(NCHUNK, C)
    idx = jnp.take(idx, _PERM_J, axis=0).reshape(NW, CPW, KSUB, SUB)
    return _gather_k(fused, idx)
